# baseline (speedup 1.0000x reference)
import functools
import math

import jax
import jax.numpy as jnp
from jax import lax
from jax.experimental import pallas as pl
from jax.experimental.pallas import tpu as pltpu
from jax.experimental.pallas import tpu_sc as plsc

N = 4096
C = 768
E = 64
K = 2
H = 3072
CAP = 160

TB = 512
HB = 512
NT = N // TB
NH = H // HB


def _gelu(v):
    return 0.5 * v * (1.0 + jax.lax.erf(v * (1.0 / math.sqrt(2.0))))


def _router_body(x_ref, wr_ref, br_ref, eb_ref, idx_ref, w_ref):
    x = x_ref[...]
    z = jnp.dot(x, wr_ref[...], preferred_element_type=jnp.float32)
    z = z + br_ref[0, :][None, :]
    s = jax.nn.sigmoid(z) + eb_ref[0, :][None, :]
    logits = jnp.log(jnp.clip(s, 1e-12, None))
    m = jnp.max(logits, axis=-1, keepdims=True)
    ex = jnp.exp(logits - m)
    probs = ex / jnp.sum(ex, axis=-1, keepdims=True)

    iota = jax.lax.broadcasted_iota(jnp.int32, probs.shape, 1)
    big = jnp.int32(E)
    m1 = jnp.max(probs, axis=-1, keepdims=True)
    i1 = jnp.min(jnp.where(probs == m1, iota, big), axis=-1, keepdims=True)
    masked = jnp.where(iota == i1, -jnp.inf, probs)
    m2 = jnp.max(masked, axis=-1, keepdims=True)
    i2 = jnp.min(jnp.where(masked == m2, iota, big), axis=-1, keepdims=True)

    tot = jnp.clip(m1 + m2, 1e-12, None)
    idx_ref[...] = jnp.concatenate([i1, i2], axis=1)
    w_ref[...] = jnp.concatenate([m1 / tot, m2 / tot], axis=1)


def _router(x, Wr, br, eb):
    return pl.pallas_call(
        _router_body,
        grid=(NT,),
        in_specs=[
            pl.BlockSpec((TB, C), lambda t: (t, 0)),
            pl.BlockSpec((C, E), lambda t: (0, 0)),
            pl.BlockSpec((1, E), lambda t: (0, 0)),
            pl.BlockSpec((1, E), lambda t: (0, 0)),
        ],
        out_specs=[
            pl.BlockSpec((TB, K), lambda t: (t, 0)),
            pl.BlockSpec((TB, K), lambda t: (t, 0)),
        ],
        out_shape=[
            jax.ShapeDtypeStruct((N, K), jnp.int32),
            jax.ShapeDtypeStruct((N, K), jnp.float32),
        ],
    )(x, Wr, br.reshape(1, E), eb.reshape(1, E))


def _expert_body(xe_ref, w1_ref, b1_ref, w2_ref, b2_ref, wt_ref, out_ref):
    hb = pl.program_id(1)

    @pl.when(hb == 0)
    def _():
        out_ref[...] = jnp.broadcast_to(b2_ref[0, 0, :][None, None, :],
                                        out_ref.shape)

    xe = xe_ref[0]
    u = jnp.dot(xe, w1_ref[0], preferred_element_type=jnp.float32)
    h = _gelu(u + b1_ref[0, 0, :][None, :])
    out_ref[...] += jnp.dot(h, w2_ref[0],
                            preferred_element_type=jnp.float32)[None]

    @pl.when(hb == NH - 1)
    def _():
        out_ref[...] = out_ref[...] * wt_ref[0, 0, :][None, :, None]


def _expert_ffn(xe, W1, b1, W2, b2, w_eff_pad):
    ec = lambda e: jnp.minimum(e, E - 1)
    return pl.pallas_call(
        _expert_body,
        grid=(E + 1, NH),
        in_specs=[
            pl.BlockSpec((1, CAP, C), lambda e, h: (ec(e), 0, 0)),
            pl.BlockSpec((1, C, HB), lambda e, h: (ec(e), 0, h)),
            pl.BlockSpec((1, 1, HB), lambda e, h: (ec(e), 0, h)),
            pl.BlockSpec((1, HB, C), lambda e, h: (ec(e), h, 0)),
            pl.BlockSpec((1, 1, C), lambda e, h: (ec(e), 0, 0)),
            pl.BlockSpec((1, 1, CAP), lambda e, h: (e, 0, 0)),
        ],
        out_specs=pl.BlockSpec((1, CAP, C), lambda e, h: (e, 0, 0)),
        out_shape=jax.ShapeDtypeStruct((E + 1, CAP, C), jnp.float32),
        compiler_params=pltpu.CompilerParams(
            dimension_semantics=("arbitrary", "arbitrary"),
        ),
    )(xe, W1, b1.reshape(E, 1, H), W2, b2.reshape(E, 1, C), w_eff_pad)


def _shared_body(x_ref, w1_ref, b1_ref, w2_ref, b2_ref, out_ref):
    hb = pl.program_id(1)

    @pl.when(hb == 0)
    def _():
        out_ref[...] = jnp.broadcast_to(b2_ref[0, :][None, :], out_ref.shape)

    u = jnp.dot(x_ref[...], w1_ref[...], preferred_element_type=jnp.float32)
    h = _gelu(u + b1_ref[0, :][None, :])
    out_ref[...] += jnp.dot(h, w2_ref[...], preferred_element_type=jnp.float32)


def _shared_ffn(x, Ws1, bs1, Ws2, bs2):
    return pl.pallas_call(
        _shared_body,
        grid=(NT, NH),
        in_specs=[
            pl.BlockSpec((TB, C), lambda t, h: (t, 0)),
            pl.BlockSpec((C, HB), lambda t, h: (0, h)),
            pl.BlockSpec((1, HB), lambda t, h: (0, h)),
            pl.BlockSpec((HB, C), lambda t, h: (h, 0)),
            pl.BlockSpec((1, C), lambda t, h: (0, 0)),
        ],
        out_specs=pl.BlockSpec((TB, C), lambda t, h: (t, 0)),
        out_shape=jax.ShapeDtypeStruct((N, C), jnp.float32),
        compiler_params=pltpu.CompilerParams(
            dimension_semantics=("arbitrary", "arbitrary"),
        ),
    )(x, Ws1, bs1.reshape(1, H), Ws2, bs2.reshape(1, C))


_SC_MESH = plsc.VectorSubcoreMesh(core_axis_name="c", subcore_axis_name="s")
_NW = 32
_RPW = (E * CAP) // _NW
_GCH = 4
_GROWS = _RPW // _GCH
_NP = N * K
_NPV = _NP // 16
_ML = N
_MLV = _ML // 16
_SINK = _NP
_INV_PAD = 128


def _sc_dispatch_gather(x, pe_flat, pw_flat):
    @functools.partial(
        pl.kernel,
        mesh=_SC_MESH,
        out_type=[
            jax.ShapeDtypeStruct((E * CAP, C), jnp.float32),
            jax.ShapeDtypeStruct((E * CAP,), jnp.float32),
            jax.ShapeDtypeStruct((_NP + _INV_PAD,), jnp.int32),
        ],
        scratch_types=[
            pltpu.VMEM((_NP,), jnp.int32),
            pltpu.VMEM((_NP,), jnp.float32),
            pltpu.VMEM((_ML + 16,), jnp.float32),
            pltpu.VMEM((_ML + 16,), jnp.int32),
            pltpu.VMEM((_ML,), jnp.int32),
            pltpu.VMEM((_ML // 128, 128), jnp.int32),
            pltpu.VMEM((_ML // 128, 128), jnp.int32),
            pltpu.VMEM((2 * CAP,), jnp.int32),
            pltpu.VMEM((2 * CAP,), jnp.float32),
            pltpu.VMEM((_GROWS, C), jnp.float32),
            pltpu.SemaphoreType.DMA,
        ],
        compiler_params=pltpu.CompilerParams(needs_layout_passes=False),
    )
    def dispatch_k(x_hbm, pe_hbm, pw_hbm, xe_hbm, bw_hbm, inv_hbm,
                   pe_v, pw_v, wl, pidl, rk, pid2, rk2, stok, swt,
                   rows_v, sem):
        wid = lax.axis_index("s") * 2 + lax.axis_index("c")
        lanes = jax.lax.iota(jnp.int32, 16)
        pltpu.sync_copy(pe_hbm, pe_v)
        pltpu.sync_copy(pw_hbm, pw_v)

        for eo in range(2):
            e = 2 * wid + eo
            ef = e.astype(jnp.float32)

            def init_body(jb, _):
                pid2[jb // 8, pl.ds((jb % 8) * 16, 16)] = jnp.full(
                    (16,), _SINK, jnp.int32)
                return 0
            lax.fori_loop(0, _MLV, init_body, 0)

            def scan_body(jb, off):
                s = pl.ds(jb * 16, 16)
                mask = pe_v[s] == e
                off_s = off[0]
                plsc.store_compressed(wl.at[pl.ds(off_s, 16)], pw_v[s], mask=mask)
                plsc.store_compressed(pidl.at[pl.ds(off_s, 16)],
                                      lanes + jb * 16, mask=mask)
                return off + plsc.all_reduce_population_count(mask)
            off = lax.fori_loop(0, _NPV, scan_body,
                                jnp.zeros((16,), jnp.int32))
            m_s = off[0]
            nvb = (m_s + 15) // 16

            def arrival(_):
                def ab(jb, _):
                    rk[pl.ds(jb * 16, 16)] = lanes + jb * 16
                    return 0
                return lax.fori_loop(0, nvb, ab, 0)

            def byweight(_):
                def outer(ib, _):
                    si = pl.ds(ib * 16, 16)
                    ki = ef * 2.0 - wl[si]
                    pi = pidl[si]
                    def inner(jb, cnt):
                        u = wl[pl.ds(jb * 16, 16)]
                        for l in range(16):
                            kj = ef * 2.0 - u[l]
                            pj = jb * 16 + l
                            hit = (kj < ki) | ((kj == ki) & (pj < pi))
                            hit = hit & (pj < m_s)
                            cnt = cnt + jnp.where(hit, 1, 0)
                        return cnt
                    rk[si] = lax.fori_loop(0, nvb, inner,
                                           jnp.zeros((16,), jnp.int32))
                    return 0
                return lax.fori_loop(0, nvb, outer, 0)

            lax.cond(m_s > CAP, byweight, arrival, 0)

            for jb in range(CAP // 16):
                z = pl.ds(eo * CAP + jb * 16, 16)
                stok[z] = jnp.zeros((16,), jnp.int32)
                swt[z] = jnp.zeros((16,), jnp.float32)

            def emit(jb, _):
                s = pl.ds(jb * 16, 16)
                rkv = rk[s]
                valid = (lanes + jb * 16) < off
                kept = valid & (rkv < CAP)
                tokv = jax.lax.shift_right_logical(pidl[s], 1)
                plsc.store_scatter(stok, [rkv + eo * CAP], tokv, mask=kept)
                plsc.store_scatter(swt, [rkv + eo * CAP], wl[s], mask=kept)
                r2 = pl.ds((jb % 8) * 16, 16)
                pid2[jb // 8, r2] = jnp.where(valid, pidl[s], _SINK)
                rk2[jb // 8, r2] = jnp.where(kept, e * CAP + rkv, E * CAP)
                return 0
            lax.fori_loop(0, nvb, emit, 0)

            def inv_scatter(cc, _):
                pltpu.async_copy(rk2.at[cc], inv_hbm.at[pid2.at[cc]],
                                 sem).wait()
                return 0
            lax.fori_loop(0, (m_s + 127) // 128, inv_scatter, 0)
            pltpu.sync_copy(swt.at[pl.ds(eo * CAP, CAP)],
                            bw_hbm.at[pl.ds(e * CAP, CAP)])

        base = wid * _RPW
        for ch in range(_GCH):
            pltpu.async_copy(
                x_hbm.at[stok.at[pl.ds(ch * _GROWS, _GROWS)]],
                rows_v, sem).wait()
            pltpu.sync_copy(rows_v,
                            xe_hbm.at[pl.ds(base + ch * _GROWS, _GROWS)])

    return dispatch_k(x, pe_flat, pw_flat)


_CTOK = 32
_TPW = N // _NW
_CCH = _TPW // _CTOK
_NV = C // 16


def _sc_combine(oew, inv_flat, shared):
    @functools.partial(
        pl.kernel,
        mesh=_SC_MESH,
        out_type=jax.ShapeDtypeStruct((N, C), jnp.float32),
        scratch_types=[
            pltpu.VMEM((2 * _CTOK,), jnp.int32),
            pltpu.VMEM((2 * _CTOK, C), jnp.float32),
            pltpu.VMEM((_CTOK, C), jnp.float32),
            pltpu.SemaphoreType.DMA,
        ],
    )
    def combine_k(oew_hbm, inv_hbm, sh_hbm, out_hbm,
                  inv_v, rows_v, acc_v, sem):
        wid = lax.axis_index("s") * 2 + lax.axis_index("c")
        t0 = wid * _TPW
        for ch in range(_CCH):
            tb = t0 + ch * _CTOK
            pltpu.sync_copy(inv_hbm.at[pl.ds(2 * tb, 2 * _CTOK)], inv_v)
            pltpu.async_copy(oew_hbm.at[inv_v], rows_v, sem).wait()
            pltpu.sync_copy(sh_hbm.at[pl.ds(tb, _CTOK)], acc_v)

            def tok_body(t, _):
                def vreg_body(j, _):
                    s = pl.ds(j * 16, 16)
                    acc_v[t, s] = (acc_v[t, s] + rows_v[2 * t, s]
                                   + rows_v[2 * t + 1, s])
                    return 0

                return lax.fori_loop(0, _NV, vreg_body, 0)

            lax.fori_loop(0, _CTOK, tok_body, 0)
            pltpu.sync_copy(acc_v, out_hbm.at[pl.ds(tb, _CTOK)])

    return combine_k(oew, inv_flat, shared)


def kernel(x, Wr, br, expert_bias, W1, b1, W2, b2, Ws1, bs1, Ws2, bs2):
    idx, w = _router(x, Wr, br, expert_bias)
    xe_flat, buf_w, inv_raw = _sc_dispatch_gather(
        x, idx.reshape(-1), w.reshape(-1))
    w_eff_pad = jnp.concatenate(
        [buf_w.reshape(E, 1, CAP), jnp.zeros((1, 1, CAP), x.dtype)], axis=0)
    shared = _shared_ffn(x, Ws1, bs1, Ws2, bs2)
    oe = _expert_ffn(xe_flat.reshape(E, CAP, C), W1, b1, W2, b2, w_eff_pad)
    return _sc_combine(oe.reshape((E + 1) * CAP, C), inv_raw[:N * K], shared)


# speedup vs baseline: 1.0034x; 1.0034x over previous
import functools
import math

import jax
import jax.numpy as jnp
from jax import lax
from jax.experimental import pallas as pl
from jax.experimental.pallas import tpu as pltpu
from jax.experimental.pallas import tpu_sc as plsc

N = 4096
C = 768
E = 64
K = 2
H = 3072
CAP = 160

TB = 512
HB = 512
NT = N // TB
NH = H // HB


def _gelu(v):
    return 0.5 * v * (1.0 + jax.lax.erf(v * (1.0 / math.sqrt(2.0))))


def _router_body(x_ref, wr_ref, br_ref, eb_ref, idx_ref, w_ref):
    x = x_ref[...]
    z = jnp.dot(x, wr_ref[...], preferred_element_type=jnp.float32)
    z = z + br_ref[0, :][None, :]
    s = jax.nn.sigmoid(z) + eb_ref[0, :][None, :]
    logits = jnp.log(jnp.clip(s, 1e-12, None))
    m = jnp.max(logits, axis=-1, keepdims=True)
    ex = jnp.exp(logits - m)
    probs = ex / jnp.sum(ex, axis=-1, keepdims=True)

    iota = jax.lax.broadcasted_iota(jnp.int32, probs.shape, 1)
    big = jnp.int32(E)
    m1 = jnp.max(probs, axis=-1, keepdims=True)
    i1 = jnp.min(jnp.where(probs == m1, iota, big), axis=-1, keepdims=True)
    masked = jnp.where(iota == i1, -jnp.inf, probs)
    m2 = jnp.max(masked, axis=-1, keepdims=True)
    i2 = jnp.min(jnp.where(masked == m2, iota, big), axis=-1, keepdims=True)

    tot = jnp.clip(m1 + m2, 1e-12, None)
    idx_ref[...] = jnp.concatenate([i1, i2], axis=1)
    w_ref[...] = jnp.concatenate([m1 / tot, m2 / tot], axis=1)


def _router(x, Wr, br, eb):
    return pl.pallas_call(
        _router_body,
        grid=(NT,),
        in_specs=[
            pl.BlockSpec((TB, C), lambda t: (t, 0)),
            pl.BlockSpec((C, E), lambda t: (0, 0)),
            pl.BlockSpec((1, E), lambda t: (0, 0)),
            pl.BlockSpec((1, E), lambda t: (0, 0)),
        ],
        out_specs=[
            pl.BlockSpec((TB, K), lambda t: (t, 0)),
            pl.BlockSpec((TB, K), lambda t: (t, 0)),
        ],
        out_shape=[
            jax.ShapeDtypeStruct((N, K), jnp.int32),
            jax.ShapeDtypeStruct((N, K), jnp.float32),
        ],
    )(x, Wr, br.reshape(1, E), eb.reshape(1, E))


def _expert_body(xe_ref, w1_ref, b1_ref, w2_ref, b2_ref, wt_ref, out_ref):
    hb = pl.program_id(1)

    @pl.when(hb == 0)
    def _():
        out_ref[...] = jnp.broadcast_to(b2_ref[0, 0, :][None, None, :],
                                        out_ref.shape)

    xe = xe_ref[0]
    u = jnp.dot(xe, w1_ref[0], preferred_element_type=jnp.float32)
    h = _gelu(u + b1_ref[0, 0, :][None, :])
    out_ref[...] += jnp.dot(h, w2_ref[0],
                            preferred_element_type=jnp.float32)[None]

    @pl.when(hb == NH - 1)
    def _():
        out_ref[...] = out_ref[...] * wt_ref[0, 0, :][None, :, None]


def _expert_ffn(xe, W1, b1, W2, b2, w_eff_pad):
    ec = lambda e: jnp.minimum(e, E - 1)
    return pl.pallas_call(
        _expert_body,
        grid=(E + 1, NH),
        in_specs=[
            pl.BlockSpec((1, CAP, C), lambda e, h: (ec(e), 0, 0)),
            pl.BlockSpec((1, C, HB), lambda e, h: (ec(e), 0, h)),
            pl.BlockSpec((1, 1, HB), lambda e, h: (ec(e), 0, h)),
            pl.BlockSpec((1, HB, C), lambda e, h: (ec(e), h, 0)),
            pl.BlockSpec((1, 1, C), lambda e, h: (ec(e), 0, 0)),
            pl.BlockSpec((1, 1, CAP), lambda e, h: (e, 0, 0)),
        ],
        out_specs=pl.BlockSpec((1, CAP, C), lambda e, h: (e, 0, 0)),
        out_shape=jax.ShapeDtypeStruct((E + 1, CAP, C), jnp.float32),
        compiler_params=pltpu.CompilerParams(
            dimension_semantics=("arbitrary", "arbitrary"),
        ),
    )(xe, W1, b1.reshape(E, 1, H), W2, b2.reshape(E, 1, C), w_eff_pad)


def _shared_body(x_ref, w1_ref, b1_ref, w2_ref, b2_ref, out_ref):
    hb = pl.program_id(1)

    @pl.when(hb == 0)
    def _():
        out_ref[...] = jnp.broadcast_to(b2_ref[0, :][None, :], out_ref.shape)

    u = jnp.dot(x_ref[...], w1_ref[...], preferred_element_type=jnp.float32)
    h = _gelu(u + b1_ref[0, :][None, :])
    out_ref[...] += jnp.dot(h, w2_ref[...], preferred_element_type=jnp.float32)


def _shared_ffn(x, Ws1, bs1, Ws2, bs2):
    return pl.pallas_call(
        _shared_body,
        grid=(NT, NH),
        in_specs=[
            pl.BlockSpec((TB, C), lambda t, h: (t, 0)),
            pl.BlockSpec((C, HB), lambda t, h: (0, h)),
            pl.BlockSpec((1, HB), lambda t, h: (0, h)),
            pl.BlockSpec((HB, C), lambda t, h: (h, 0)),
            pl.BlockSpec((1, C), lambda t, h: (0, 0)),
        ],
        out_specs=pl.BlockSpec((TB, C), lambda t, h: (t, 0)),
        out_shape=jax.ShapeDtypeStruct((N, C), jnp.float32),
        compiler_params=pltpu.CompilerParams(
            dimension_semantics=("arbitrary", "arbitrary"),
        ),
    )(x, Ws1, bs1.reshape(1, H), Ws2, bs2.reshape(1, C))


_SC_MESH = plsc.VectorSubcoreMesh(core_axis_name="c", subcore_axis_name="s")
_NW = 32
_RPW = (E * CAP) // _NW
_GCH = 8
_GROWS = _RPW // _GCH
_NP = N * K
_NPV = _NP // 16
_ML = N
_MLV = _ML // 16
_SINK = _NP
_INV_PAD = 128


def _sc_dispatch_gather(x, pe_flat, pw_flat):
    @functools.partial(
        pl.kernel,
        mesh=_SC_MESH,
        out_type=[
            jax.ShapeDtypeStruct((E * CAP, C), jnp.float32),
            jax.ShapeDtypeStruct((E * CAP,), jnp.float32),
            jax.ShapeDtypeStruct((_NP + _INV_PAD,), jnp.int32),
        ],
        scratch_types=[
            pltpu.VMEM((_NP,), jnp.int32),
            pltpu.VMEM((_NP,), jnp.float32),
            pltpu.VMEM((_ML + 16,), jnp.float32),
            pltpu.VMEM((_ML + 16,), jnp.int32),
            pltpu.VMEM((_ML + 16,), jnp.float32),
            pltpu.VMEM((_ML + 16,), jnp.int32),
            pltpu.VMEM((_ML,), jnp.int32),
            pltpu.VMEM((_ML // 128, 128), jnp.int32),
            pltpu.VMEM((_ML // 128, 128), jnp.int32),
            pltpu.VMEM((2 * CAP,), jnp.int32),
            pltpu.VMEM((2 * CAP,), jnp.float32),
            pltpu.VMEM((_GROWS, C), jnp.float32),
            pltpu.VMEM((_GROWS, C), jnp.float32),
            pltpu.SemaphoreType.DMA,
            pltpu.SemaphoreType.DMA,
            pltpu.SemaphoreType.DMA,
            pltpu.SemaphoreType.DMA,
        ],
        compiler_params=pltpu.CompilerParams(needs_layout_passes=False),
    )
    def dispatch_k(x_hbm, pe_hbm, pw_hbm, xe_hbm, bw_hbm, inv_hbm,
                   pe_v, pw_v, wl0, pid0, wl1, pid1, rk, pid2, rk2,
                   stok, swt, rowsa, rowsb, sga, sgb, swa, swb):
        wid = lax.axis_index("s") * 2 + lax.axis_index("c")
        lanes = jax.lax.iota(jnp.int32, 16)
        pltpu.sync_copy(pe_hbm, pe_v)
        pltpu.sync_copy(pw_hbm, pw_v)
        e0 = 2 * wid
        e1 = 2 * wid + 1

        def scan_chunk(ob, carry):
            off0, off1 = carry
            for ib in range(16):
                jb = ob * 16 + ib
                s = pl.ds(jb * 16, 16)
                ev = pe_v[s]
                wv = pw_v[s]
                pidv = lanes + jb * 16
                m0 = ev == e0
                m1 = ev == e1
                plsc.store_compressed(wl0.at[pl.ds(off0, 16)], wv, mask=m0)
                plsc.store_compressed(pid0.at[pl.ds(off0, 16)], pidv, mask=m0)
                plsc.store_compressed(wl1.at[pl.ds(off1, 16)], wv, mask=m1)
                plsc.store_compressed(pid1.at[pl.ds(off1, 16)], pidv, mask=m1)
                off0 = off0 + plsc.all_reduce_population_count(m0)[0]
                off1 = off1 + plsc.all_reduce_population_count(m1)[0]
            return off0, off1

        z32 = jnp.int32(0)
        off0, off1 = lax.fori_loop(0, _NPV // 16, scan_chunk, (z32, z32))

        for eo, e, wl, pidl, m_s in ((0, e0, wl0, pid0, off0),
                                     (1, e1, wl1, pid1, off1)):
            ef = e.astype(jnp.float32)
            nvb = (m_s + 15) // 16

            sinkv = jnp.full((16,), _SINK, jnp.int32)
            for r in range(_ML // 128):
                for cc in range(8):
                    pid2[r, pl.ds(cc * 16, 16)] = sinkv

            def arrival(_):
                def ab(jb, _):
                    rk[pl.ds(jb * 16, 16)] = lanes + jb * 16
                    return 0
                return lax.fori_loop(0, nvb, ab, 0)

            def byweight(_):
                def outer(ib, _):
                    si = pl.ds(ib * 16, 16)
                    ki = ef * 2.0 - wl[si]
                    pi = pidl[si]
                    def inner(jb, cnt):
                        u = wl[pl.ds(jb * 16, 16)]
                        for l in range(16):
                            kj = ef * 2.0 - u[l]
                            pj = jb * 16 + l
                            hit = (kj < ki) | ((kj == ki) & (pj < pi))
                            hit = hit & (pj < m_s)
                            cnt = cnt + jnp.where(hit, 1, 0)
                        return cnt
                    rk[si] = lax.fori_loop(0, nvb, inner,
                                           jnp.zeros((16,), jnp.int32))
                    return 0
                return lax.fori_loop(0, nvb, outer, 0)

            lax.cond(m_s > CAP, byweight, arrival, 0)

            zi = jnp.zeros((16,), jnp.int32)
            zf = jnp.zeros((16,), jnp.float32)
            for jb in range(CAP // 16):
                z = pl.ds(eo * CAP + jb * 16, 16)
                stok[z] = zi
                swt[z] = zf

            def emit(jb, _):
                s = pl.ds(jb * 16, 16)
                rkv = rk[s]
                valid = (lanes + jb * 16) < m_s
                kept = valid & (rkv < CAP)
                tokv = jax.lax.shift_right_logical(pidl[s], 1)
                plsc.store_scatter(stok, [rkv + eo * CAP], tokv, mask=kept)
                plsc.store_scatter(swt, [rkv + eo * CAP], wl[s], mask=kept)
                r2 = pl.ds((jb % 8) * 16, 16)
                pid2[jb // 8, r2] = jnp.where(valid, pidl[s], _SINK)
                rk2[jb // 8, r2] = jnp.where(kept, e * CAP + rkv, E * CAP)
                return 0
            lax.fori_loop(0, nvb, emit, 0)

            def inv_scatter(cc, _):
                pltpu.async_copy(rk2.at[cc], inv_hbm.at[pid2.at[cc]],
                                 sga).wait()
                return 0
            lax.fori_loop(0, (m_s + 127) // 128, inv_scatter, 0)
            pltpu.sync_copy(swt.at[pl.ds(eo * CAP, CAP)],
                            bw_hbm.at[pl.ds(e * CAP, CAP)])

        base = wid * _RPW
        bufs = (rowsa, rowsb)
        gsems = (sga, sgb)
        wsems = (swa, swb)

        def start_gather(ch, buf, sem):
            return pltpu.async_copy(
                x_hbm.at[stok.at[pl.ds(ch * _GROWS, _GROWS)]], buf, sem)

        def start_write(ch, buf, sem):
            return pltpu.async_copy(
                buf, xe_hbm.at[pl.ds(base + ch * _GROWS, _GROWS)], sem)

        gh = [None, None]
        wh = [None, None]
        gh[0] = start_gather(0, bufs[0], gsems[0])
        for ch in range(_GCH):
            b = ch % 2
            nb = (ch + 1) % 2
            if ch + 1 < _GCH:
                if wh[nb] is not None:
                    wh[nb].wait()
                gh[nb] = start_gather(ch + 1, bufs[nb], gsems[nb])
            gh[b].wait()
            wh[b] = start_write(ch, bufs[b], wsems[b])
        wh[0].wait()
        wh[1].wait()

    return dispatch_k(x, pe_flat, pw_flat)


_CTOK = 32
_TPW = N // _NW
_CCH = _TPW // _CTOK
_NV = C // 16


def _sc_combine(oew, inv_flat, shared):
    @functools.partial(
        pl.kernel,
        mesh=_SC_MESH,
        out_type=jax.ShapeDtypeStruct((N, C), jnp.float32),
        scratch_types=[
            pltpu.VMEM((2 * _CTOK,), jnp.int32),
            pltpu.VMEM((2 * _CTOK, C), jnp.float32),
            pltpu.VMEM((_CTOK, C), jnp.float32),
            pltpu.SemaphoreType.DMA,
        ],
    )
    def combine_k(oew_hbm, inv_hbm, sh_hbm, out_hbm,
                  inv_v, rows_v, acc_v, sem):
        wid = lax.axis_index("s") * 2 + lax.axis_index("c")
        t0 = wid * _TPW
        for ch in range(_CCH):
            tb = t0 + ch * _CTOK
            pltpu.sync_copy(inv_hbm.at[pl.ds(2 * tb, 2 * _CTOK)], inv_v)
            pltpu.async_copy(oew_hbm.at[inv_v], rows_v, sem).wait()
            pltpu.sync_copy(sh_hbm.at[pl.ds(tb, _CTOK)], acc_v)

            def tok_body(t, _):
                def vreg_body(j, _):
                    s = pl.ds(j * 16, 16)
                    acc_v[t, s] = (acc_v[t, s] + rows_v[2 * t, s]
                                   + rows_v[2 * t + 1, s])
                    return 0

                return lax.fori_loop(0, _NV, vreg_body, 0)

            lax.fori_loop(0, _CTOK, tok_body, 0)
            pltpu.sync_copy(acc_v, out_hbm.at[pl.ds(tb, _CTOK)])

    return combine_k(oew, inv_flat, shared)


def kernel(x, Wr, br, expert_bias, W1, b1, W2, b2, Ws1, bs1, Ws2, bs2):
    idx, w = _router(x, Wr, br, expert_bias)
    xe_flat, buf_w, inv_raw = _sc_dispatch_gather(
        x, idx.reshape(-1), w.reshape(-1))
    w_eff_pad = jnp.concatenate(
        [buf_w.reshape(E, 1, CAP), jnp.zeros((1, 1, CAP), x.dtype)], axis=0)
    shared = _shared_ffn(x, Ws1, bs1, Ws2, bs2)
    oe = _expert_ffn(xe_flat.reshape(E, CAP, C), W1, b1, W2, b2, w_eff_pad)
    return _sc_combine(oe.reshape((E + 1) * CAP, C), inv_raw[:N * K], shared)


# speedup vs baseline: 1.0056x; 1.0021x over previous
import functools
import math

import jax
import jax.numpy as jnp
from jax import lax
from jax.experimental import pallas as pl
from jax.experimental.pallas import tpu as pltpu
from jax.experimental.pallas import tpu_sc as plsc

N = 4096
C = 768
E = 64
K = 2
H = 3072
CAP = 160

TB = 512
HB = 512
NT = N // TB
NH = H // HB


def _gelu(v):
    return 0.5 * v * (1.0 + jax.lax.erf(v * (1.0 / math.sqrt(2.0))))


def _router_body(x_ref, wr_ref, br_ref, eb_ref, idx_ref, w_ref):
    x = x_ref[...]
    z = jnp.dot(x, wr_ref[...], preferred_element_type=jnp.float32)
    z = z + br_ref[0, :][None, :]
    s = jax.nn.sigmoid(z) + eb_ref[0, :][None, :]
    logits = jnp.log(jnp.clip(s, 1e-12, None))
    m = jnp.max(logits, axis=-1, keepdims=True)
    ex = jnp.exp(logits - m)
    probs = ex / jnp.sum(ex, axis=-1, keepdims=True)

    iota = jax.lax.broadcasted_iota(jnp.int32, probs.shape, 1)
    big = jnp.int32(E)
    m1 = jnp.max(probs, axis=-1, keepdims=True)
    i1 = jnp.min(jnp.where(probs == m1, iota, big), axis=-1, keepdims=True)
    masked = jnp.where(iota == i1, -jnp.inf, probs)
    m2 = jnp.max(masked, axis=-1, keepdims=True)
    i2 = jnp.min(jnp.where(masked == m2, iota, big), axis=-1, keepdims=True)

    tot = jnp.clip(m1 + m2, 1e-12, None)
    idx_ref[...] = jnp.concatenate([i1, i2], axis=1)
    w_ref[...] = jnp.concatenate([m1 / tot, m2 / tot], axis=1)


def _router(x, Wr, br, eb):
    return pl.pallas_call(
        _router_body,
        grid=(NT,),
        in_specs=[
            pl.BlockSpec((TB, C), lambda t: (t, 0)),
            pl.BlockSpec((C, E), lambda t: (0, 0)),
            pl.BlockSpec((1, E), lambda t: (0, 0)),
            pl.BlockSpec((1, E), lambda t: (0, 0)),
        ],
        out_specs=[
            pl.BlockSpec((TB, K), lambda t: (t, 0)),
            pl.BlockSpec((TB, K), lambda t: (t, 0)),
        ],
        out_shape=[
            jax.ShapeDtypeStruct((N, K), jnp.int32),
            jax.ShapeDtypeStruct((N, K), jnp.float32),
        ],
    )(x, Wr, br.reshape(1, E), eb.reshape(1, E))


def _expert_body(xe_ref, w1_ref, b1_ref, w2_ref, b2_ref, wt_ref, out_ref):
    hb = pl.program_id(1)

    @pl.when(hb == 0)
    def _():
        out_ref[...] = jnp.broadcast_to(b2_ref[0, 0, :][None, None, :],
                                        out_ref.shape)

    xe = xe_ref[0]
    u = jnp.dot(xe, w1_ref[0], preferred_element_type=jnp.float32)
    h = _gelu(u + b1_ref[0, 0, :][None, :])
    out_ref[...] += jnp.dot(h, w2_ref[0],
                            preferred_element_type=jnp.float32)[None]

    @pl.when(hb == NH - 1)
    def _():
        out_ref[...] = out_ref[...] * wt_ref[0, 0, :][None, :, None]


def _expert_ffn(xe, W1, b1, W2, b2, w_eff_pad):
    ec = lambda e: jnp.minimum(e, E - 1)
    return pl.pallas_call(
        _expert_body,
        grid=(E + 1, NH),
        in_specs=[
            pl.BlockSpec((1, CAP, C), lambda e, h: (ec(e), 0, 0)),
            pl.BlockSpec((1, C, HB), lambda e, h: (ec(e), 0, h)),
            pl.BlockSpec((1, 1, HB), lambda e, h: (ec(e), 0, h)),
            pl.BlockSpec((1, HB, C), lambda e, h: (ec(e), h, 0)),
            pl.BlockSpec((1, 1, C), lambda e, h: (ec(e), 0, 0)),
            pl.BlockSpec((1, 1, CAP), lambda e, h: (e, 0, 0)),
        ],
        out_specs=pl.BlockSpec((1, CAP, C), lambda e, h: (e, 0, 0)),
        out_shape=jax.ShapeDtypeStruct((E + 1, CAP, C), jnp.float32),
        compiler_params=pltpu.CompilerParams(
            dimension_semantics=("arbitrary", "arbitrary"),
        ),
    )(xe, W1, b1.reshape(E, 1, H), W2, b2.reshape(E, 1, C), w_eff_pad)


def _shared_body(x_ref, w1_ref, b1_ref, w2_ref, b2_ref, out_ref):
    hb = pl.program_id(1)

    @pl.when(hb == 0)
    def _():
        out_ref[...] = jnp.broadcast_to(b2_ref[0, :][None, :], out_ref.shape)

    u = jnp.dot(x_ref[...], w1_ref[...], preferred_element_type=jnp.float32)
    h = _gelu(u + b1_ref[0, :][None, :])
    out_ref[...] += jnp.dot(h, w2_ref[...], preferred_element_type=jnp.float32)


def _shared_ffn(x, Ws1, bs1, Ws2, bs2):
    return pl.pallas_call(
        _shared_body,
        grid=(NT, NH),
        in_specs=[
            pl.BlockSpec((TB, C), lambda t, h: (t, 0)),
            pl.BlockSpec((C, HB), lambda t, h: (0, h)),
            pl.BlockSpec((1, HB), lambda t, h: (0, h)),
            pl.BlockSpec((HB, C), lambda t, h: (h, 0)),
            pl.BlockSpec((1, C), lambda t, h: (0, 0)),
        ],
        out_specs=pl.BlockSpec((TB, C), lambda t, h: (t, 0)),
        out_shape=jax.ShapeDtypeStruct((N, C), jnp.float32),
        compiler_params=pltpu.CompilerParams(
            dimension_semantics=("arbitrary", "arbitrary"),
        ),
    )(x, Ws1, bs1.reshape(1, H), Ws2, bs2.reshape(1, C))


_SC_MESH = plsc.VectorSubcoreMesh(core_axis_name="c", subcore_axis_name="s")
_NW = 32
_RPW = (E * CAP) // _NW
_GCH = 8
_GROWS = _RPW // _GCH
_NP = N * K
_NPV = _NP // 16
_ML = N
_MLV = _ML // 16
_SINK = _NP
_INV_PAD = 128


def _sc_dispatch_gather(x, pe_flat, pw_flat):
    @functools.partial(
        pl.kernel,
        mesh=_SC_MESH,
        out_type=[
            jax.ShapeDtypeStruct((E * CAP, C), jnp.float32),
            jax.ShapeDtypeStruct((E * CAP,), jnp.float32),
            jax.ShapeDtypeStruct((_NP + _INV_PAD,), jnp.int32),
        ],
        scratch_types=[
            pltpu.VMEM((_NP,), jnp.int32),
            pltpu.VMEM((_NP,), jnp.float32),
            pltpu.VMEM((_ML + 16,), jnp.float32),
            pltpu.VMEM((_ML + 16,), jnp.int32),
            pltpu.VMEM((_ML + 16,), jnp.float32),
            pltpu.VMEM((_ML + 16,), jnp.int32),
            pltpu.VMEM((_ML,), jnp.int32),
            pltpu.VMEM((_ML // 128, 128), jnp.int32),
            pltpu.VMEM((_ML // 128, 128), jnp.int32),
            pltpu.VMEM((2 * CAP,), jnp.int32),
            pltpu.VMEM((2 * CAP,), jnp.float32),
            pltpu.VMEM((_GROWS, C), jnp.float32),
            pltpu.VMEM((_GROWS, C), jnp.float32),
            pltpu.SemaphoreType.DMA,
            pltpu.SemaphoreType.DMA,
            pltpu.SemaphoreType.DMA,
            pltpu.SemaphoreType.DMA,
        ],
        compiler_params=pltpu.CompilerParams(needs_layout_passes=False),
    )
    def dispatch_k(x_hbm, pe_hbm, pw_hbm, xe_hbm, bw_hbm, inv_hbm,
                   pe_v, pw_v, wl0, pid0, wl1, pid1, rk, pid2, rk2,
                   stok, swt, rowsa, rowsb, sga, sgb, swa, swb):
        wid = lax.axis_index("s") * 2 + lax.axis_index("c")
        lanes = jax.lax.iota(jnp.int32, 16)
        pltpu.sync_copy(pe_hbm, pe_v)
        pltpu.sync_copy(pw_hbm, pw_v)
        e0 = 2 * wid
        e1 = 2 * wid + 1

        def scan_chunk(ob, carry):
            off0, off1 = carry
            for ib in range(16):
                jb = ob * 16 + ib
                s = pl.ds(jb * 16, 16)
                ev = pe_v[s]
                wv = pw_v[s]
                pidv = lanes + jb * 16
                m0 = ev == e0
                m1 = ev == e1
                plsc.store_compressed(wl0.at[pl.ds(off0, 16)], wv, mask=m0)
                plsc.store_compressed(pid0.at[pl.ds(off0, 16)], pidv, mask=m0)
                plsc.store_compressed(wl1.at[pl.ds(off1, 16)], wv, mask=m1)
                plsc.store_compressed(pid1.at[pl.ds(off1, 16)], pidv, mask=m1)
                off0 = off0 + plsc.all_reduce_population_count(m0)[0]
                off1 = off1 + plsc.all_reduce_population_count(m1)[0]
            return off0, off1

        z32 = jnp.int32(0)
        off0, off1 = lax.fori_loop(0, _NPV // 16, scan_chunk, (z32, z32))

        for eo, e, wl, pidl, m_s in ((0, e0, wl0, pid0, off0),
                                     (1, e1, wl1, pid1, off1)):
            ef = e.astype(jnp.float32)
            nvb = (m_s + 15) // 16

            sinkv = jnp.full((16,), _SINK, jnp.int32)
            for r in range(_ML // 128):
                for cc in range(8):
                    pid2[r, pl.ds(cc * 16, 16)] = sinkv

            def arrival(_):
                def ab(jb, _):
                    rk[pl.ds(jb * 16, 16)] = lanes + jb * 16
                    return 0
                return lax.fori_loop(0, nvb, ab, 0)

            def byweight(_):
                def outer(ib, _):
                    si = pl.ds(ib * 16, 16)
                    ki = ef * 2.0 - wl[si]
                    pi = lanes + ib * 16
                    def inner(jb, cnt):
                        u = wl[pl.ds(jb * 16, 16)]
                        for l in range(16):
                            kj = ef * 2.0 - u[l]
                            pj = jb * 16 + l
                            hit = (kj < ki) | ((kj == ki) & (pj < pi))
                            hit = hit & (pj < m_s)
                            cnt = cnt + jnp.where(hit, 1, 0)
                        return cnt
                    rk[si] = lax.fori_loop(0, nvb, inner,
                                           jnp.zeros((16,), jnp.int32))
                    return 0
                return lax.fori_loop(0, nvb, outer, 0)

            lax.cond(m_s > CAP, byweight, arrival, 0)

            zi = jnp.zeros((16,), jnp.int32)
            zf = jnp.zeros((16,), jnp.float32)
            for jb in range(CAP // 16):
                z = pl.ds(eo * CAP + jb * 16, 16)
                stok[z] = zi
                swt[z] = zf

            def emit(jb, _):
                s = pl.ds(jb * 16, 16)
                rkv = rk[s]
                valid = (lanes + jb * 16) < m_s
                kept = valid & (rkv < CAP)
                tokv = jax.lax.shift_right_logical(pidl[s], 1)
                plsc.store_scatter(stok, [rkv + eo * CAP], tokv, mask=kept)
                plsc.store_scatter(swt, [rkv + eo * CAP], wl[s], mask=kept)
                r2 = pl.ds((jb % 8) * 16, 16)
                pid2[jb // 8, r2] = jnp.where(valid, pidl[s], _SINK)
                rk2[jb // 8, r2] = jnp.where(kept, e * CAP + rkv, E * CAP)
                return 0
            lax.fori_loop(0, nvb, emit, 0)

            def inv_scatter(cc, _):
                pltpu.async_copy(rk2.at[cc], inv_hbm.at[pid2.at[cc]],
                                 sga).wait()
                return 0
            lax.fori_loop(0, (m_s + 127) // 128, inv_scatter, 0)
            pltpu.sync_copy(swt.at[pl.ds(eo * CAP, CAP)],
                            bw_hbm.at[pl.ds(e * CAP, CAP)])

        base = wid * _RPW
        bufs = (rowsa, rowsb)
        gsems = (sga, sgb)
        wsems = (swa, swb)

        def start_gather(ch, buf, sem):
            return pltpu.async_copy(
                x_hbm.at[stok.at[pl.ds(ch * _GROWS, _GROWS)]], buf, sem)

        def start_write(ch, buf, sem):
            return pltpu.async_copy(
                buf, xe_hbm.at[pl.ds(base + ch * _GROWS, _GROWS)], sem)

        gh = [None, None]
        wh = [None, None]
        gh[0] = start_gather(0, bufs[0], gsems[0])
        for ch in range(_GCH):
            b = ch % 2
            nb = (ch + 1) % 2
            if ch + 1 < _GCH:
                if wh[nb] is not None:
                    wh[nb].wait()
                gh[nb] = start_gather(ch + 1, bufs[nb], gsems[nb])
            gh[b].wait()
            wh[b] = start_write(ch, bufs[b], wsems[b])
        wh[0].wait()
        wh[1].wait()

    return dispatch_k(x, pe_flat, pw_flat)


_CTOK = 32
_TPW = N // _NW
_CCH = _TPW // _CTOK
_NV = C // 16


def _sc_combine(oew, inv_flat, shared):
    @functools.partial(
        pl.kernel,
        mesh=_SC_MESH,
        out_type=jax.ShapeDtypeStruct((N, C), jnp.float32),
        scratch_types=[
            pltpu.VMEM((2 * _CTOK,), jnp.int32),
            pltpu.VMEM((2 * _CTOK, C), jnp.float32),
            pltpu.VMEM((_CTOK, C), jnp.float32),
            pltpu.SemaphoreType.DMA,
        ],
    )
    def combine_k(oew_hbm, inv_hbm, sh_hbm, out_hbm,
                  inv_v, rows_v, acc_v, sem):
        wid = lax.axis_index("s") * 2 + lax.axis_index("c")
        t0 = wid * _TPW
        for ch in range(_CCH):
            tb = t0 + ch * _CTOK
            pltpu.sync_copy(inv_hbm.at[pl.ds(2 * tb, 2 * _CTOK)], inv_v)
            pltpu.async_copy(oew_hbm.at[inv_v], rows_v, sem).wait()
            pltpu.sync_copy(sh_hbm.at[pl.ds(tb, _CTOK)], acc_v)

            def tok_body(t, _):
                def vreg_body(j, _):
                    s = pl.ds(j * 16, 16)
                    acc_v[t, s] = (acc_v[t, s] + rows_v[2 * t, s]
                                   + rows_v[2 * t + 1, s])
                    return 0

                return lax.fori_loop(0, _NV, vreg_body, 0)

            lax.fori_loop(0, _CTOK, tok_body, 0)
            pltpu.sync_copy(acc_v, out_hbm.at[pl.ds(tb, _CTOK)])

    return combine_k(oew, inv_flat, shared)


def kernel(x, Wr, br, expert_bias, W1, b1, W2, b2, Ws1, bs1, Ws2, bs2):
    idx, w = _router(x, Wr, br, expert_bias)
    xe_flat, buf_w, inv_raw = _sc_dispatch_gather(
        x, idx.reshape(-1), w.reshape(-1))
    w_eff_pad = jnp.concatenate(
        [buf_w.reshape(E, 1, CAP), jnp.zeros((1, 1, CAP), x.dtype)], axis=0)
    shared = _shared_ffn(x, Ws1, bs1, Ws2, bs2)
    oe = _expert_ffn(xe_flat.reshape(E, CAP, C), W1, b1, W2, b2, w_eff_pad)
    return _sc_combine(oe.reshape((E + 1) * CAP, C), inv_raw[:N * K], shared)
